# Initial kernel scaffold; baseline (speedup 1.0000x reference)
import jax
import jax.numpy as jnp
from jax.experimental import pallas as pl


def kernel(h, edge_index, edge_sh, edge_features, params):
    raise NotImplementedError("write your pallas kernel here")


# speedup vs baseline: 1.2162x; 1.2162x over previous
import functools

import jax
import jax.numpy as jnp
import numpy as np
from jax import lax
from jax.experimental import pallas as pl
from jax.experimental.pallas import tpu as pltpu
from jax.experimental.pallas import tpu_sc as plsc

MUL0_ = 64
MUL1_ = 32
N_ = 10000
E_ = 320000
TABW = 448
ACCW = 176
NC = 2
NS = 16
LANES = 16
EPT = E_ // (NC * NS)
CH = 16
NCHUNK = EPT // CH
NPAD = 10240
RPT = NPAD // NS


def _table_body(h_ref, w_ref, o_ref):
    o_ref[...] = jnp.dot(h_ref[...], w_ref[...],
                         preferred_element_type=jnp.float32)


def _build_table(h, wbig):
    return pl.pallas_call(
        _table_body,
        grid=(5,),
        in_specs=[
            pl.BlockSpec((N_ // 5, 160), lambda i: (i, 0)),
            pl.BlockSpec((160, TABW), lambda i: (0, 0)),
        ],
        out_specs=pl.BlockSpec((N_ // 5, TABW), lambda i: (i, 0)),
        out_shape=jax.ShapeDtypeStruct((N_, TABW), jnp.float32),
    )(h, wbig)


def _silu(x):
    return x * (1.0 / (1.0 + jnp.exp(-x)))


def _edge_body(ef_ref, sh_ref, w1, b1, w2, b2, w3, b3, t_ref):
    f = _silu(jnp.dot(ef_ref[...], w1[...],
                      preferred_element_type=jnp.float32) + b1[...])
    f = _silu(jnp.dot(f, w2[...],
                      preferred_element_type=jnp.float32) + b2[...])
    s = jnp.dot(f, w3[...], preferred_element_type=jnp.float32) + b3[...]
    t_ref[...] = (s * sh_ref[...]).T


def _edge_t(ef, esh, p):
    B = 3200
    return pl.pallas_call(
        _edge_body,
        grid=(E_ // B,),
        in_specs=[
            pl.BlockSpec((B, 16), lambda i: (i, 0)),
            pl.BlockSpec((B, 4), lambda i: (i, 0)),
            pl.BlockSpec((16, 64), lambda i: (0, 0)),
            pl.BlockSpec((1, 64), lambda i: (0, 0)),
            pl.BlockSpec((64, 64), lambda i: (0, 0)),
            pl.BlockSpec((1, 64), lambda i: (0, 0)),
            pl.BlockSpec((64, 1), lambda i: (0, 0)),
            pl.BlockSpec((1, 1), lambda i: (0, 0)),
        ],
        out_specs=pl.BlockSpec((4, B), lambda i: (0, i)),
        out_shape=jax.ShapeDtypeStruct((4, E_), jnp.float32),
    )(ef, esh, p['mw1'], p['mb1'].reshape(1, 64), p['mw2'],
      p['mb2'].reshape(1, 64), p['mw3'], p['mb3'].reshape(1, 1))


def _lane_splat(vec, e):
    idx = jnp.full((LANES, 1), e, jnp.int32)
    return lax.gather(
        vec, idx,
        lax.GatherDimensionNumbers(offset_dims=(), collapsed_slice_dims=(0,),
                                   start_index_map=(0,)),
        (1,), mode=lax.GatherScatterMode.PROMISE_IN_BOUNDS)


def _sc_body(tab_ref, ei_ref, t_ref, out_ref,
             src_v, dst_v, t_v, rows_v, pay_v, acc, gsem):
    c = lax.axis_index("c")
    s = lax.axis_index("s")
    wid = s * NC + c
    zvec = jnp.zeros((LANES,), jnp.float32)
    iota16 = lax.broadcasted_iota(jnp.int32, (LANES,), 0)

    for e in range(CH):
        for cc in range(ACCW // LANES):
            pay_v[e, pl.ds(cc * LANES, LANES)] = zvec

    def zcp(j, carry):
        pltpu.sync_copy(pay_v, acc.at[pl.ds(s * RPT + j * CH, CH)])
        return carry
    lax.fori_loop(0, RPT // CH, zcp, 0)
    plsc.subcore_barrier()

    cntv = jnp.where(iota16 == 0, 1.0, 0.0).astype(jnp.float32)
    for e in range(CH):
        pay_v[e, pl.ds(160, LANES)] = cntv

    base_e = wid * EPT

    def chunk(i, carry):
        eb = base_e + i * CH
        pltpu.sync_copy(ei_ref.at[0, pl.ds(eb, CH)], src_v)
        pltpu.sync_copy(ei_ref.at[1, pl.ds(eb, CH)], dst_v)
        for k in range(4):
            pltpu.sync_copy(t_ref.at[k, pl.ds(eb, CH)], t_v.at[k])
        pltpu.async_copy(tab_ref.at[src_v], rows_v, gsem).wait()
        tv0 = t_v[0, pl.ds(0, LANES)]
        tv1 = t_v[1, pl.ds(0, LANES)]
        tv2 = t_v[2, pl.ds(0, LANES)]
        tv3 = t_v[3, pl.ds(0, LANES)]
        for e in range(CH):
            t0b = _lane_splat(tv0, e)
            t1b = _lane_splat(tv1, e)
            t2b = _lane_splat(tv2, e)
            t3b = _lane_splat(tv3, e)
            for j in range(4):
                a = t0b * rows_v[e, pl.ds(j * 16, LANES)]
                a = a + t1b * rows_v[e, pl.ds(64 + j * 16, LANES)]
                a = a + t2b * rows_v[e, pl.ds(128 + j * 16, LANES)]
                a = a + t3b * rows_v[e, pl.ds(192 + j * 16, LANES)]
                pay_v[e, pl.ds(j * 16, LANES)] = a
            for v in range(6):
                md = (iota16 + 16 * v) % 3
                tpat = jnp.where(md == 0, t1b,
                                 jnp.where(md == 1, t2b, t3b))
                m1 = tpat * rows_v[e, pl.ds(256 + v * 16, LANES)] \
                    + t0b * rows_v[e, pl.ds(352 + v * 16, LANES)]
                pay_v[e, pl.ds(64 + v * 16, LANES)] = m1
        pltpu.sync_copy(pay_v, acc.at[dst_v], add=True)
        return carry

    lax.fori_loop(0, NCHUNK, chunk, 0)
    plsc.subcore_barrier()

    pltpu.sync_copy(acc.at[pl.ds(s * RPT, RPT)],
                    out_ref.at[c, pl.ds(s * RPT, RPT)])


def _sc_scatter(table, edge_index, t):
    mesh = plsc.VectorSubcoreMesh(core_axis_name="c", subcore_axis_name="s")
    kfn = pl.kernel(
        _sc_body,
        out_type=jax.ShapeDtypeStruct((NC, NPAD, ACCW), jnp.float32),
        mesh=mesh,
        scratch_types=[
            pltpu.VMEM((CH,), jnp.int32),
            pltpu.VMEM((CH,), jnp.int32),
            pltpu.VMEM((4, CH), jnp.float32),
            pltpu.VMEM((CH, TABW), jnp.float32),
            pltpu.VMEM((CH, ACCW), jnp.float32),
            pltpu.VMEM_SHARED((NPAD, ACCW), jnp.float32),
            pltpu.SemaphoreType.DMA,
        ],
        compiler_params=pltpu.CompilerParams(use_tc_tiling_on_sc=False),
    )
    return kfn(table, edge_index, t)


def _fin_body(acc_ref, h_ref, wsi_ref, g0, b0, g1, o_ref):
    sums = acc_ref[0, :N_, :] + acc_ref[1, :N_, :]
    cnt = jnp.maximum(sums[:, 160:161], 1.0)
    agg = sums[:, :160] / cnt
    out = agg + jnp.dot(h_ref[...], wsi_ref[...],
                        preferred_element_type=jnp.float32)
    sc = out[:, :MUL0_]
    mu = jnp.mean(sc, axis=0, keepdims=True)
    xc = sc - mu
    var = jnp.mean(xc * xc, axis=0, keepdims=True)
    scn = xc * lax.rsqrt(var + 1e-5) * g0[...] + b0[...]
    v = out[:, MUL0_:]
    colsum = jnp.sum(v * v, axis=0, keepdims=True)
    r = lax.broadcasted_iota(jnp.int32, (96, 32), 0)
    cix = lax.broadcasted_iota(jnp.int32, (96, 32), 1)
    S = (r // 3 == cix).astype(jnp.float32)
    fn = jnp.dot(colsum, S, preferred_element_type=jnp.float32) / N_
    scale32 = g1[...] * lax.rsqrt(fn + 1e-5)
    r2 = lax.broadcasted_iota(jnp.int32, (32, 96), 0)
    c2 = lax.broadcasted_iota(jnp.int32, (32, 96), 1)
    S2 = (c2 // 3 == r2).astype(jnp.float32)
    scale96 = jnp.dot(scale32, S2, preferred_element_type=jnp.float32)
    vout = v * scale96
    o_ref[...] = jnp.concatenate([scn, vout], axis=1) + h_ref[...]


def _finish(acc, h, wsi, g0, b0, g1):
    return pl.pallas_call(
        _fin_body,
        in_specs=[
            pl.BlockSpec((NC, NPAD, ACCW), lambda: (0, 0, 0)),

            pl.BlockSpec((N_, 160), lambda: (0, 0)),
            pl.BlockSpec((160, 160), lambda: (0, 0)),
            pl.BlockSpec((1, 64), lambda: (0, 0)),
            pl.BlockSpec((1, 64), lambda: (0, 0)),
            pl.BlockSpec((1, 32), lambda: (0, 0)),
        ],
        out_specs=pl.BlockSpec((N_, 160), lambda: (0, 0)),
        out_shape=jax.ShapeDtypeStruct((N_, 160), jnp.float32),
    )(acc, h, wsi, g0.reshape(1, 64), b0.reshape(1, 64), g1.reshape(1, 32))


def _weights(p):
    A1 = 1.0 / np.sqrt(MUL0_)
    A2 = 1.0 / np.sqrt(MUL0_)
    A3 = 1.0 / np.sqrt(MUL1_)
    A4 = 1.0 / np.sqrt(MUL1_ * 3.0)
    eye3 = jnp.eye(3, dtype=jnp.float32)
    zb = A4 * jnp.einsum('de,uj->udej', eye3, p['W4']).reshape(96, 192)
    y2b = A2 * jnp.repeat(p['W2'], 3, axis=1)
    y3b = A3 * jnp.einsum('uk,de->udke', p['W3'], eye3).reshape(96, 96)
    top = jnp.concatenate(
        [A1 * p['W1'], jnp.zeros((64, 192), jnp.float32), y2b,
         jnp.zeros((64, 96), jnp.float32)], axis=1)
    bot = jnp.concatenate(
        [jnp.zeros((96, 64), jnp.float32), zb,
         jnp.zeros((96, 96), jnp.float32), y3b], axis=1)
    wbig = jnp.concatenate([top, bot], axis=0)
    ws1 = jnp.einsum('uk,de->udke', p['Ws1'], eye3).reshape(96, 96)
    wsi = jnp.block(
        [[p['Ws0'] / np.sqrt(MUL0_), jnp.zeros((64, 96), jnp.float32)],
         [jnp.zeros((96, 64), jnp.float32), ws1 / np.sqrt(MUL1_)]])
    return wbig, wsi


@jax.jit
def kernel(h, edge_index, edge_sh, edge_features, params):
    wbig, wsi = _weights(params)
    table = _build_table(h, wbig)
    t = _edge_t(edge_features, edge_sh, params)
    acc = _sc_scatter(table, edge_index, t)
    return _finish(acc, h, wsi, params['g0'], params['b0'], params['g1'])


# speedup vs baseline: 2.5801x; 2.1215x over previous
import functools

import jax
import jax.numpy as jnp
import numpy as np
from jax import lax
from jax.experimental import pallas as pl
from jax.experimental.pallas import tpu as pltpu
from jax.experimental.pallas import tpu_sc as plsc

MUL0_ = 64
MUL1_ = 32
N_ = 10000
E_ = 320000
TABW = 384
ACCW = 168
NC = 2
NS = 16
LANES = 16
EPT = E_ // (NC * NS)
CH = 16
SUP = 400
NSUP = EPT // SUP
NPAD = 10240
RPT = NPAD // NS


def _table_body(h_ref, w_ref, o_ref):
    o_ref[...] = jnp.dot(h_ref[...], w_ref[...],
                         preferred_element_type=jnp.float32)


def _build_table(h, wbig):
    return pl.pallas_call(
        _table_body,
        grid=(5,),
        in_specs=[
            pl.BlockSpec((N_ // 5, 160), lambda i: (i, 0)),
            pl.BlockSpec((160, TABW), lambda i: (0, 0)),
        ],
        out_specs=pl.BlockSpec((N_ // 5, TABW), lambda i: (i, 0)),
        out_shape=jax.ShapeDtypeStruct((N_, TABW), jnp.float32),
    )(h, wbig)


def _silu(x):
    return x * (1.0 / (1.0 + jnp.exp(-x)))


def _edge_body(ef_ref, sh_ref, w1, b1, w2, b2, w3, b3, t_ref):
    f = _silu(jnp.dot(ef_ref[...], w1[...],
                      preferred_element_type=jnp.float32) + b1[...])
    f = _silu(jnp.dot(f, w2[...],
                      preferred_element_type=jnp.float32) + b2[...])
    s = jnp.dot(f, w3[...], preferred_element_type=jnp.float32) + b3[...]
    t_ref[...] = (s * sh_ref[...]).T


def _edge_t(ef, esh, p):
    B = 3200
    return pl.pallas_call(
        _edge_body,
        grid=(E_ // B,),
        in_specs=[
            pl.BlockSpec((B, 16), lambda i: (i, 0)),
            pl.BlockSpec((B, 4), lambda i: (i, 0)),
            pl.BlockSpec((16, 64), lambda i: (0, 0)),
            pl.BlockSpec((1, 64), lambda i: (0, 0)),
            pl.BlockSpec((64, 64), lambda i: (0, 0)),
            pl.BlockSpec((1, 64), lambda i: (0, 0)),
            pl.BlockSpec((64, 1), lambda i: (0, 0)),
            pl.BlockSpec((1, 1), lambda i: (0, 0)),
        ],
        out_specs=pl.BlockSpec((4, B), lambda i: (0, i)),
        out_shape=jax.ShapeDtypeStruct((4, E_), jnp.float32),
    )(ef, esh, p['mw1'], p['mb1'].reshape(1, 64), p['mw2'],
      p['mb2'].reshape(1, 64), p['mw3'], p['mb3'].reshape(1, 1))


def _lane_splat(vec, e):
    idx = jnp.full((LANES, 1), e, jnp.int32)
    return lax.gather(
        vec, idx,
        lax.GatherDimensionNumbers(offset_dims=(), collapsed_slice_dims=(0,),
                                   start_index_map=(0,)),
        (1,), mode=lax.GatherScatterMode.PROMISE_IN_BOUNDS)


def _sc_body(tab_ref, ei_ref, t_ref, out_ref,
             src_v, dst_v, t_v, rows_a, rows_b, pay_a, pay_b, acc,
             gsem_a, gsem_b):
    c = lax.axis_index("c")
    s = lax.axis_index("s")
    wid = s * NC + c
    zvec = jnp.zeros((LANES,), jnp.float32)
    iota16 = lax.broadcasted_iota(jnp.int32, (LANES,), 0)

    for e in range(CH):
        for cc in range(ACCW // LANES):
            pay_a[e, pl.ds(cc * LANES, LANES)] = zvec

    def zcp(j, carry):
        pltpu.sync_copy(pay_a, acc.at[pl.ds(s * RPT + j * CH, CH)])
        return carry
    lax.fori_loop(0, RPT // CH, zcp, 0)
    plsc.subcore_barrier()

    cntv = jnp.where(iota16 == 0, 1.0, 0.0).astype(jnp.float32)
    base_e = wid * EPT

    def wait(rows_v, sem):
        pltpu.make_async_copy(tab_ref.at[pl.ds(0, CH)], rows_v, sem).wait()

    def super_body(sp, carry):
        eb = base_e + sp * SUP
        pltpu.sync_copy(ei_ref.at[0, pl.ds(eb, SUP)], src_v)
        pltpu.sync_copy(ei_ref.at[1, pl.ds(eb, SUP)], dst_v)
        for k in range(4):
            pltpu.sync_copy(t_ref.at[k, pl.ds(eb, SUP)], t_v.at[k])

        def fire_dyn(cidx, rows_v, sem):
            pltpu.async_copy(
                tab_ref.at[src_v.at[pl.ds(cidx * CH, CH)]], rows_v, sem)

        def pair_body(j, carry2):
            c0 = 2 * j
            fire_dyn(c0 + 1, rows_b, gsem_b)
            wait(rows_a, gsem_a)
            compute_dyn(c0, rows_a, pay_a)
            fire_dyn(c0 + 2, rows_a, gsem_a)
            wait(rows_b, gsem_b)
            compute_dyn(c0 + 1, rows_b, pay_b)
            return carry2

        def compute_dyn(cidx, rows_v, pay_v):
            tb = cidx * CH
            tv0 = t_v[0, pl.ds(tb, LANES)]
            tv1 = t_v[1, pl.ds(tb, LANES)]
            tv2 = t_v[2, pl.ds(tb, LANES)]
            tv3 = t_v[3, pl.ds(tb, LANES)]

            def edge(e, carry3):
                t0b = _lane_splat(tv0, e)
                t1b = _lane_splat(tv1, e)
                t2b = _lane_splat(tv2, e)
                t3b = _lane_splat(tv3, e)
                tdb = (t1b, t2b, t3b)
                pay_v[e, pl.ds(0, LANES)] = cntv
                for j in range(4):
                    a = t0b * rows_v[e, pl.ds(j * 16, LANES)]
                    a = a + t1b * rows_v[e, pl.ds(64 + j * 16, LANES)]
                    a = a + t2b * rows_v[e, pl.ds(128 + j * 16, LANES)]
                    a = a + t3b * rows_v[e, pl.ds(192 + j * 16, LANES)]
                    pay_v[e, pl.ds(8 + j * 16, LANES)] = a
                y2h = (rows_v[e, pl.ds(256, LANES)],
                       rows_v[e, pl.ds(272, LANES)])
                for d in range(3):
                    for hf in range(2):
                        m1 = tdb[d] * y2h[hf] + t0b * rows_v[
                            e, pl.ds(288 + d * 32 + hf * 16, LANES)]
                        pay_v[e, pl.ds(72 + d * 32 + hf * 16, LANES)] = m1
                return carry3

            lax.fori_loop(0, CH, edge, 0)
            pltpu.sync_copy(pay_v,
                            acc.at[dst_v.at[pl.ds(cidx * CH, CH)]],
                            add=True)

        fire_dyn(0, rows_a, gsem_a)
        lax.fori_loop(0, (SUP // CH) // 2, pair_body, 0)
        wait(rows_a, gsem_a)
        compute_dyn(SUP // CH - 1, rows_a, pay_a)
        return carry

    lax.fori_loop(0, NSUP, super_body, 0)
    plsc.subcore_barrier()

    def dump(r, carry):
        pltpu.sync_copy(acc.at[pl.ds(s * RPT + r * 64, 64)],
                        out_ref.at[c, pl.ds(s * RPT + r * 64, 64)])
        return carry
    lax.fori_loop(0, RPT // 64, dump, 0)


def _sc_scatter(table, edge_index, t):
    mesh = plsc.VectorSubcoreMesh(core_axis_name="c", subcore_axis_name="s")
    kfn = pl.kernel(
        _sc_body,
        out_type=jax.ShapeDtypeStruct((NC, NPAD, ACCW), jnp.float32),
        mesh=mesh,
        scratch_types=[
            pltpu.VMEM((SUP,), jnp.int32),
            pltpu.VMEM((SUP,), jnp.int32),
            pltpu.VMEM((4, SUP), jnp.float32),
            pltpu.VMEM((CH, TABW), jnp.float32),
            pltpu.VMEM((CH, TABW), jnp.float32),
            pltpu.VMEM((CH, ACCW), jnp.float32),
            pltpu.VMEM((CH, ACCW), jnp.float32),
            pltpu.VMEM_SHARED((NPAD, ACCW), jnp.float32),
            pltpu.SemaphoreType.DMA,
            pltpu.SemaphoreType.DMA,
        ],
        compiler_params=pltpu.CompilerParams(use_tc_tiling_on_sc=False),
    )
    return kfn(table, edge_index, t)


def _fin_body(acc_ref, h_ref, wsi_ref, g0, b0, g1, o_ref):
    sums = acc_ref[0, :N_, :] + acc_ref[1, :N_, :]
    cnt = jnp.maximum(sums[:, 0:1], 1.0)
    agg = sums[:, 8:168] / cnt
    ri = lax.broadcasted_iota(jnp.int32, (96, 96), 0)
    ci = lax.broadcasted_iota(jnp.int32, (96, 96), 1)
    P = ((ri % 32) * 3 + ri // 32 == ci).astype(jnp.float32)
    agg1 = jnp.dot(agg[:, 64:160], P, preferred_element_type=jnp.float32)
    agg = jnp.concatenate([agg[:, :64], agg1], axis=1)
    out = agg + jnp.dot(h_ref[...], wsi_ref[...],
                        preferred_element_type=jnp.float32)
    sc = out[:, :MUL0_]
    mu = jnp.mean(sc, axis=0, keepdims=True)
    xc = sc - mu
    var = jnp.mean(xc * xc, axis=0, keepdims=True)
    scn = xc * lax.rsqrt(var + 1e-5) * g0[...] + b0[...]
    v = out[:, MUL0_:]
    colsum = jnp.sum(v * v, axis=0, keepdims=True)
    r = lax.broadcasted_iota(jnp.int32, (96, 32), 0)
    cix = lax.broadcasted_iota(jnp.int32, (96, 32), 1)
    S = (r // 3 == cix).astype(jnp.float32)
    fn = jnp.dot(colsum, S, preferred_element_type=jnp.float32) / N_
    scale32 = g1[...] * lax.rsqrt(fn + 1e-5)
    r2 = lax.broadcasted_iota(jnp.int32, (32, 96), 0)
    c2 = lax.broadcasted_iota(jnp.int32, (32, 96), 1)
    S2 = (c2 // 3 == r2).astype(jnp.float32)
    scale96 = jnp.dot(scale32, S2, preferred_element_type=jnp.float32)
    vout = v * scale96
    o_ref[...] = jnp.concatenate([scn, vout], axis=1) + h_ref[...]


def _finish(acc, h, wsi, g0, b0, g1):
    return pl.pallas_call(
        _fin_body,
        in_specs=[
            pl.BlockSpec((NC, NPAD, ACCW), lambda: (0, 0, 0)),

            pl.BlockSpec((N_, 160), lambda: (0, 0)),
            pl.BlockSpec((160, 160), lambda: (0, 0)),
            pl.BlockSpec((1, 64), lambda: (0, 0)),
            pl.BlockSpec((1, 64), lambda: (0, 0)),
            pl.BlockSpec((1, 32), lambda: (0, 0)),
        ],
        out_specs=pl.BlockSpec((N_, 160), lambda: (0, 0)),
        out_shape=jax.ShapeDtypeStruct((N_, 160), jnp.float32),
    )(acc, h, wsi, g0.reshape(1, 64), b0.reshape(1, 64), g1.reshape(1, 32))


def _weights(p):
    A1 = 1.0 / np.sqrt(MUL0_)
    A2 = 1.0 / np.sqrt(MUL0_)
    A3 = 1.0 / np.sqrt(MUL1_)
    A4 = 1.0 / np.sqrt(MUL1_ * 3.0)
    eye3 = jnp.eye(3, dtype=jnp.float32)
    zb = A4 * jnp.einsum('de,uj->udej', eye3, p['W4']).reshape(96, 192)
    y2b = A2 * p['W2']
    y3b = A3 * jnp.einsum('uk,de->udek', p['W3'], eye3).reshape(96, 96)
    top = jnp.concatenate(
        [A1 * p['W1'], jnp.zeros((64, 192), jnp.float32), y2b,
         jnp.zeros((64, 96), jnp.float32)], axis=1)
    bot = jnp.concatenate(
        [jnp.zeros((96, 64), jnp.float32), zb,
         jnp.zeros((96, 32), jnp.float32), y3b], axis=1)
    wbig = jnp.concatenate([top, bot], axis=0)
    ws1 = jnp.einsum('uk,de->udke', p['Ws1'], eye3).reshape(96, 96)
    wsi = jnp.block(
        [[p['Ws0'] / np.sqrt(MUL0_), jnp.zeros((64, 96), jnp.float32)],
         [jnp.zeros((96, 64), jnp.float32), ws1 / np.sqrt(MUL1_)]])
    return wbig, wsi


@jax.jit
def kernel(h, edge_index, edge_sh, edge_features, params):
    wbig, wsi = _weights(params)
    table = _build_table(h, wbig)
    t = _edge_t(edge_features, edge_sh, params)
    acc = _sc_scatter(table, edge_index, t)
    return _finish(acc, h, wsi, params['g0'], params['b0'], params['g1'])


# speedup vs baseline: 2.7708x; 1.0739x over previous
import functools

import jax
import jax.numpy as jnp
import numpy as np
from jax import lax
from jax.experimental import pallas as pl
from jax.experimental.pallas import tpu as pltpu
from jax.experimental.pallas import tpu_sc as plsc

MUL0_ = 64
MUL1_ = 32
N_ = 10000
E_ = 320000
TABW = 384
ACCW = 168
NC = 2
NS = 16
LANES = 16
EPT = E_ // (NC * NS)
CH = 16
SUP = 400
NSUP = EPT // SUP
NPAD = 10240
RPT = NPAD // NS


def _table_body(h_ref, w_ref, o_ref):
    o_ref[...] = jnp.dot(h_ref[...], w_ref[...],
                         preferred_element_type=jnp.float32)


def _build_table(h, wbig):
    return pl.pallas_call(
        _table_body,
        grid=(5,),
        in_specs=[
            pl.BlockSpec((N_ // 5, 160), lambda i: (i, 0)),
            pl.BlockSpec((160, TABW), lambda i: (0, 0)),
        ],
        out_specs=pl.BlockSpec((N_ // 5, TABW), lambda i: (i, 0)),
        out_shape=jax.ShapeDtypeStruct((N_, TABW), jnp.float32),
    )(h, wbig)


def _silu(x):
    return x * (1.0 / (1.0 + jnp.exp(-x)))


def _edge_body(ef_ref, sh_ref, w1, b1, w2, b2, w3, b3, t_ref):
    f = _silu(jnp.dot(ef_ref[...], w1[...],
                      preferred_element_type=jnp.float32) + b1[...])
    f = _silu(jnp.dot(f, w2[...],
                      preferred_element_type=jnp.float32) + b2[...])
    s = jnp.dot(f, w3[...], preferred_element_type=jnp.float32) + b3[...]
    t_ref[...] = (s * sh_ref[...]).T


def _edge_t(ef, esh, p):
    B = 3200
    return pl.pallas_call(
        _edge_body,
        grid=(E_ // B,),
        in_specs=[
            pl.BlockSpec((B, 16), lambda i: (i, 0)),
            pl.BlockSpec((B, 4), lambda i: (i, 0)),
            pl.BlockSpec((16, 64), lambda i: (0, 0)),
            pl.BlockSpec((1, 64), lambda i: (0, 0)),
            pl.BlockSpec((64, 64), lambda i: (0, 0)),
            pl.BlockSpec((1, 64), lambda i: (0, 0)),
            pl.BlockSpec((64, 1), lambda i: (0, 0)),
            pl.BlockSpec((1, 1), lambda i: (0, 0)),
        ],
        out_specs=pl.BlockSpec((4, B), lambda i: (0, i)),
        out_shape=jax.ShapeDtypeStruct((4, E_), jnp.float32),
    )(ef, esh, p['mw1'], p['mb1'].reshape(1, 64), p['mw2'],
      p['mb2'].reshape(1, 64), p['mw3'], p['mb3'].reshape(1, 1))


def _lane_splat(vec, e):
    idx = jnp.full((LANES, 1), e, jnp.int32)
    return lax.gather(
        vec, idx,
        lax.GatherDimensionNumbers(offset_dims=(), collapsed_slice_dims=(0,),
                                   start_index_map=(0,)),
        (1,), mode=lax.GatherScatterMode.PROMISE_IN_BOUNDS)


def _sc_body(tab_ref, ei_ref, t_ref, out_ref,
             src_v, dst_v, t_v, rows_a, rows_b, pay_a, pay_b, acc,
             gsem_a, gsem_b, ssem_a, ssem_b):
    c = lax.axis_index("c")
    s = lax.axis_index("s")
    wid = s * NC + c
    zvec = jnp.zeros((LANES,), jnp.float32)
    iota16 = lax.broadcasted_iota(jnp.int32, (LANES,), 0)

    for pv in (pay_a, pay_b):
        for e in range(CH):
            for cc in range(ACCW // LANES):
                pv[e, pl.ds(cc * LANES, LANES)] = zvec
            pv[e, pl.ds(ACCW - LANES, LANES)] = zvec

    def zcp(j, carry):
        pltpu.sync_copy(pay_a, acc.at[pl.ds(s * RPT + j * CH, CH)])
        return carry
    lax.fori_loop(0, RPT // CH, zcp, 0)
    plsc.subcore_barrier()

    dst_v[pl.ds(0, LANES)] = iota16 + s * RPT
    pltpu.async_copy(pay_a, acc.at[dst_v.at[pl.ds(0, CH)]], ssem_a, add=True)
    pltpu.async_copy(pay_b, acc.at[dst_v.at[pl.ds(0, CH)]], ssem_b, add=True)

    cntv = jnp.where(iota16 == 0, 1.0, 0.0).astype(jnp.float32)
    base_e = wid * EPT

    def wait(rows_v, sem):
        pltpu.make_async_copy(tab_ref.at[pl.ds(0, CH)], rows_v, sem).wait()

    def super_body(sp, carry):
        eb = base_e + sp * SUP
        pltpu.sync_copy(ei_ref.at[0, pl.ds(eb, SUP)], src_v)
        pltpu.sync_copy(ei_ref.at[1, pl.ds(eb, SUP)], dst_v)
        for k in range(4):
            pltpu.sync_copy(t_ref.at[k, pl.ds(eb, SUP)], t_v.at[k])

        def fire_dyn(cidx, rows_v, sem):
            pltpu.async_copy(
                tab_ref.at[src_v.at[pl.ds(cidx * CH, CH)]], rows_v, sem)

        def pair_body(j, carry2):
            c0 = 2 * j
            fire_dyn(c0 + 1, rows_b, gsem_b)
            wait(rows_a, gsem_a)
            compute_dyn(c0, rows_a, pay_a, ssem_a)
            fire_dyn(c0 + 2, rows_a, gsem_a)
            wait(rows_b, gsem_b)
            compute_dyn(c0 + 1, rows_b, pay_b, ssem_b)
            return carry2

        def compute_dyn(cidx, rows_v, pay_v, ssem):
            tb = cidx * CH
            tv0 = t_v[0, pl.ds(tb, LANES)]
            tv1 = t_v[1, pl.ds(tb, LANES)]
            tv2 = t_v[2, pl.ds(tb, LANES)]
            tv3 = t_v[3, pl.ds(tb, LANES)]
            pltpu.make_async_copy(out_ref.at[0, pl.ds(0, CH)], pay_v,
                                  ssem).wait()

            def edge4(it, carry3):
                for k in range(4):
                    e = it * 4 + k
                    t0b = _lane_splat(tv0, e)
                    t1b = _lane_splat(tv1, e)
                    t2b = _lane_splat(tv2, e)
                    t3b = _lane_splat(tv3, e)
                    tdb = (t1b, t2b, t3b)
                    pay_v[e, pl.ds(0, LANES)] = cntv
                    for j in range(4):
                        a = t0b * rows_v[e, pl.ds(j * 16, LANES)]
                        a = a + t1b * rows_v[e, pl.ds(64 + j * 16, LANES)]
                        a = a + t2b * rows_v[e, pl.ds(128 + j * 16, LANES)]
                        a = a + t3b * rows_v[e, pl.ds(192 + j * 16, LANES)]
                        pay_v[e, pl.ds(8 + j * 16, LANES)] = a
                    y2h = (rows_v[e, pl.ds(256, LANES)],
                           rows_v[e, pl.ds(272, LANES)])
                    for d in range(3):
                        for hf in range(2):
                            m1 = tdb[d] * y2h[hf] + t0b * rows_v[
                                e, pl.ds(288 + d * 32 + hf * 16, LANES)]
                            pay_v[e, pl.ds(72 + d * 32 + hf * 16, LANES)] = m1
                return carry3

            lax.fori_loop(0, CH // 4, edge4, 0)
            pltpu.async_copy(pay_v,
                             acc.at[dst_v.at[pl.ds(cidx * CH, CH)]],
                             ssem, add=True)

        fire_dyn(0, rows_a, gsem_a)
        lax.fori_loop(0, (SUP // CH) // 2, pair_body, 0)
        wait(rows_a, gsem_a)
        compute_dyn(SUP // CH - 1, rows_a, pay_a, ssem_a)
        return carry

    lax.fori_loop(0, NSUP, super_body, 0)
    pltpu.make_async_copy(out_ref.at[0, pl.ds(0, CH)], pay_a, ssem_a).wait()
    pltpu.make_async_copy(out_ref.at[0, pl.ds(0, CH)], pay_b, ssem_b).wait()
    plsc.subcore_barrier()

    def dump(r, carry):
        pltpu.sync_copy(acc.at[pl.ds(s * RPT + r * 64, 64)],
                        out_ref.at[c, pl.ds(s * RPT + r * 64, 64)])
        return carry
    lax.fori_loop(0, RPT // 64, dump, 0)


def _sc_scatter(table, edge_index, t):
    mesh = plsc.VectorSubcoreMesh(core_axis_name="c", subcore_axis_name="s")
    kfn = pl.kernel(
        _sc_body,
        out_type=jax.ShapeDtypeStruct((NC, NPAD, ACCW), jnp.float32),
        mesh=mesh,
        scratch_types=[
            pltpu.VMEM((SUP,), jnp.int32),
            pltpu.VMEM((SUP,), jnp.int32),
            pltpu.VMEM((4, SUP), jnp.float32),
            pltpu.VMEM((CH, TABW), jnp.float32),
            pltpu.VMEM((CH, TABW), jnp.float32),
            pltpu.VMEM((CH, ACCW), jnp.float32),
            pltpu.VMEM((CH, ACCW), jnp.float32),
            pltpu.VMEM_SHARED((NPAD, ACCW), jnp.float32),
            pltpu.SemaphoreType.DMA,
            pltpu.SemaphoreType.DMA,
            pltpu.SemaphoreType.DMA,
            pltpu.SemaphoreType.DMA,
        ],
        compiler_params=pltpu.CompilerParams(use_tc_tiling_on_sc=False),
    )
    return kfn(table, edge_index, t)


def _fin_body(acc_ref, h_ref, wsi_ref, g0, b0, g1, o_ref):
    sums = acc_ref[0, :N_, :] + acc_ref[1, :N_, :]
    cnt = jnp.maximum(sums[:, 0:1], 1.0)
    agg = sums[:, 8:168] / cnt
    ri = lax.broadcasted_iota(jnp.int32, (96, 96), 0)
    ci = lax.broadcasted_iota(jnp.int32, (96, 96), 1)
    P = ((ri % 32) * 3 + ri // 32 == ci).astype(jnp.float32)
    agg1 = jnp.dot(agg[:, 64:160], P, preferred_element_type=jnp.float32)
    agg = jnp.concatenate([agg[:, :64], agg1], axis=1)
    out = agg + jnp.dot(h_ref[...], wsi_ref[...],
                        preferred_element_type=jnp.float32)
    sc = out[:, :MUL0_]
    mu = jnp.mean(sc, axis=0, keepdims=True)
    xc = sc - mu
    var = jnp.mean(xc * xc, axis=0, keepdims=True)
    scn = xc * lax.rsqrt(var + 1e-5) * g0[...] + b0[...]
    v = out[:, MUL0_:]
    colsum = jnp.sum(v * v, axis=0, keepdims=True)
    r = lax.broadcasted_iota(jnp.int32, (96, 32), 0)
    cix = lax.broadcasted_iota(jnp.int32, (96, 32), 1)
    S = (r // 3 == cix).astype(jnp.float32)
    fn = jnp.dot(colsum, S, preferred_element_type=jnp.float32) / N_
    scale32 = g1[...] * lax.rsqrt(fn + 1e-5)
    r2 = lax.broadcasted_iota(jnp.int32, (32, 96), 0)
    c2 = lax.broadcasted_iota(jnp.int32, (32, 96), 1)
    S2 = (c2 // 3 == r2).astype(jnp.float32)
    scale96 = jnp.dot(scale32, S2, preferred_element_type=jnp.float32)
    vout = v * scale96
    o_ref[...] = jnp.concatenate([scn, vout], axis=1) + h_ref[...]


def _finish(acc, h, wsi, g0, b0, g1):
    return pl.pallas_call(
        _fin_body,
        in_specs=[
            pl.BlockSpec((NC, NPAD, ACCW), lambda: (0, 0, 0)),

            pl.BlockSpec((N_, 160), lambda: (0, 0)),
            pl.BlockSpec((160, 160), lambda: (0, 0)),
            pl.BlockSpec((1, 64), lambda: (0, 0)),
            pl.BlockSpec((1, 64), lambda: (0, 0)),
            pl.BlockSpec((1, 32), lambda: (0, 0)),
        ],
        out_specs=pl.BlockSpec((N_, 160), lambda: (0, 0)),
        out_shape=jax.ShapeDtypeStruct((N_, 160), jnp.float32),
    )(acc, h, wsi, g0.reshape(1, 64), b0.reshape(1, 64), g1.reshape(1, 32))


def _weights(p):
    A1 = 1.0 / np.sqrt(MUL0_)
    A2 = 1.0 / np.sqrt(MUL0_)
    A3 = 1.0 / np.sqrt(MUL1_)
    A4 = 1.0 / np.sqrt(MUL1_ * 3.0)
    eye3 = jnp.eye(3, dtype=jnp.float32)
    zb = A4 * jnp.einsum('de,uj->udej', eye3, p['W4']).reshape(96, 192)
    y2b = A2 * p['W2']
    y3b = A3 * jnp.einsum('uk,de->udek', p['W3'], eye3).reshape(96, 96)
    top = jnp.concatenate(
        [A1 * p['W1'], jnp.zeros((64, 192), jnp.float32), y2b,
         jnp.zeros((64, 96), jnp.float32)], axis=1)
    bot = jnp.concatenate(
        [jnp.zeros((96, 64), jnp.float32), zb,
         jnp.zeros((96, 32), jnp.float32), y3b], axis=1)
    wbig = jnp.concatenate([top, bot], axis=0)
    ws1 = jnp.einsum('uk,de->udke', p['Ws1'], eye3).reshape(96, 96)
    wsi = jnp.block(
        [[p['Ws0'] / np.sqrt(MUL0_), jnp.zeros((64, 96), jnp.float32)],
         [jnp.zeros((96, 64), jnp.float32), ws1 / np.sqrt(MUL1_)]])
    return wbig, wsi


@jax.jit
def kernel(h, edge_index, edge_sh, edge_features, params):
    wbig, wsi = _weights(params)
    table = _build_table(h, wbig)
    t = _edge_t(edge_features, edge_sh, params)
    acc = _sc_scatter(table, edge_index, t)
    return _finish(acc, h, wsi, params['g0'], params['b0'], params['g1'])


# speedup vs baseline: 2.8519x; 1.0293x over previous
import functools

import jax
import jax.numpy as jnp
import numpy as np
from jax import lax
from jax.experimental import pallas as pl
from jax.experimental.pallas import tpu as pltpu
from jax.experimental.pallas import tpu_sc as plsc

MUL0_ = 64
MUL1_ = 32
N_ = 10000
E_ = 320000
TABW = 384
ACCW = 168
NC = 2
NS = 16
LANES = 16
EPT = E_ // (NC * NS)
CH = 16
SUP = 400
NSUP = EPT // SUP
NPAD = 10240
RPT = NPAD // NS


def _table_body(h_ref, w_ref, o_ref):
    o_ref[...] = jnp.dot(h_ref[...], w_ref[...],
                         preferred_element_type=jnp.float32)


def _build_table(h, wbig):
    return pl.pallas_call(
        _table_body,
        grid=(5,),
        in_specs=[
            pl.BlockSpec((N_ // 5, 160), lambda i: (i, 0)),
            pl.BlockSpec((160, TABW), lambda i: (0, 0)),
        ],
        out_specs=pl.BlockSpec((N_ // 5, TABW), lambda i: (i, 0)),
        out_shape=jax.ShapeDtypeStruct((N_, TABW), jnp.float32),
    )(h, wbig)


def _silu(x):
    return x * (1.0 / (1.0 + jnp.exp(-x)))


def _edge_body(ef_ref, sh_ref, w1, b1, w2, b2, w3, b3, t_ref):
    f = _silu(jnp.dot(ef_ref[...], w1[...],
                      preferred_element_type=jnp.float32) + b1[...])
    f = _silu(jnp.dot(f, w2[...],
                      preferred_element_type=jnp.float32) + b2[...])
    s = jnp.dot(f, w3[...], preferred_element_type=jnp.float32) + b3[...]
    t_ref[...] = (s * sh_ref[...]).T


def _edge_t(ef, esh, p):
    B = 3200
    return pl.pallas_call(
        _edge_body,
        grid=(E_ // B,),
        in_specs=[
            pl.BlockSpec((B, 16), lambda i: (i, 0)),
            pl.BlockSpec((B, 4), lambda i: (i, 0)),
            pl.BlockSpec((16, 64), lambda i: (0, 0)),
            pl.BlockSpec((1, 64), lambda i: (0, 0)),
            pl.BlockSpec((64, 64), lambda i: (0, 0)),
            pl.BlockSpec((1, 64), lambda i: (0, 0)),
            pl.BlockSpec((64, 1), lambda i: (0, 0)),
            pl.BlockSpec((1, 1), lambda i: (0, 0)),
        ],
        out_specs=pl.BlockSpec((4, B), lambda i: (0, i)),
        out_shape=jax.ShapeDtypeStruct((4, E_), jnp.float32),
    )(ef, esh, p['mw1'], p['mb1'].reshape(1, 64), p['mw2'],
      p['mb2'].reshape(1, 64), p['mw3'], p['mb3'].reshape(1, 1))


def _lane_splat(vec, e):
    idx = jnp.full((LANES, 1), e, jnp.int32)
    return lax.gather(
        vec, idx,
        lax.GatherDimensionNumbers(offset_dims=(), collapsed_slice_dims=(0,),
                                   start_index_map=(0,)),
        (1,), mode=lax.GatherScatterMode.PROMISE_IN_BOUNDS)


def _sc_body(tab_ref, ei_ref, t_ref, out_ref,
             src_v, dst_v, t_v, rows_a, rows_b, pay_a, pay_b, acc,
             gsem_a, gsem_b, ssem_a, ssem_b):
    c = lax.axis_index("c")
    s = lax.axis_index("s")
    wid = s * NC + c
    zvec = jnp.zeros((LANES,), jnp.float32)
    iota16 = lax.broadcasted_iota(jnp.int32, (LANES,), 0)

    for pv in (pay_a, pay_b):
        for e in range(CH):
            for cc in range(ACCW // LANES):
                pv[e, pl.ds(cc * LANES, LANES)] = zvec
            pv[e, pl.ds(ACCW - LANES, LANES)] = zvec

    def zcp(j, carry):
        pltpu.async_copy(pay_a, acc.at[pl.ds(s * RPT + j * CH, CH)], gsem_a)
        return carry
    lax.fori_loop(0, RPT // CH, zcp, 0)

    def zdr(j, carry):
        pltpu.make_async_copy(out_ref.at[0, pl.ds(0, CH)], pay_a,
                              gsem_a).wait()
        return carry
    lax.fori_loop(0, RPT // CH, zdr, 0)
    plsc.subcore_barrier()

    dst_v[pl.ds(0, LANES)] = iota16 + s * RPT
    pltpu.async_copy(pay_a, acc.at[dst_v.at[pl.ds(0, CH)]], ssem_a, add=True)
    pltpu.async_copy(pay_b, acc.at[dst_v.at[pl.ds(0, CH)]], ssem_b, add=True)

    cntv = jnp.where(iota16 == 0, 1.0, 0.0).astype(jnp.float32)
    base_e = wid * EPT

    def wait(rows_v, sem):
        pltpu.make_async_copy(tab_ref.at[pl.ds(0, CH)], rows_v, sem).wait()

    def super_body(sp, carry):
        eb = base_e + sp * SUP
        pltpu.sync_copy(ei_ref.at[0, pl.ds(eb, SUP)], src_v)
        pltpu.sync_copy(ei_ref.at[1, pl.ds(eb, SUP)], dst_v)
        pltpu.sync_copy(t_ref.at[:, pl.ds(eb, SUP)], t_v)

        def fire_dyn(cidx, rows_v, sem):
            pltpu.async_copy(
                tab_ref.at[src_v.at[pl.ds(cidx * CH, CH)]], rows_v, sem)

        def pair_body(j, carry2):
            c0 = 2 * j
            fire_dyn(c0 + 1, rows_b, gsem_b)
            wait(rows_a, gsem_a)
            compute_dyn(c0, rows_a, pay_a, ssem_a)
            fire_dyn(c0 + 2, rows_a, gsem_a)
            wait(rows_b, gsem_b)
            compute_dyn(c0 + 1, rows_b, pay_b, ssem_b)
            return carry2

        def compute_dyn(cidx, rows_v, pay_v, ssem):
            tb = cidx * CH
            tv0 = t_v[0, pl.ds(tb, LANES)]
            tv1 = t_v[1, pl.ds(tb, LANES)]
            tv2 = t_v[2, pl.ds(tb, LANES)]
            tv3 = t_v[3, pl.ds(tb, LANES)]
            pltpu.make_async_copy(out_ref.at[0, pl.ds(0, CH)], pay_v,
                                  ssem).wait()

            def edge4(it, carry3):
                for k in range(4):
                    e = it * 4 + k
                    t0b = _lane_splat(tv0, e)
                    t1b = _lane_splat(tv1, e)
                    t2b = _lane_splat(tv2, e)
                    t3b = _lane_splat(tv3, e)
                    tdb = (t1b, t2b, t3b)
                    pay_v[e, pl.ds(0, LANES)] = cntv
                    for j in range(4):
                        a = (t0b * rows_v[e, pl.ds(j * 16, LANES)]
                             + t1b * rows_v[e, pl.ds(64 + j * 16, LANES)]) \
                            + (t2b * rows_v[e, pl.ds(128 + j * 16, LANES)]
                               + t3b * rows_v[e, pl.ds(192 + j * 16, LANES)])
                        pay_v[e, pl.ds(8 + j * 16, LANES)] = a
                    y2h = (rows_v[e, pl.ds(256, LANES)],
                           rows_v[e, pl.ds(272, LANES)])
                    for d in range(3):
                        for hf in range(2):
                            m1 = tdb[d] * y2h[hf] + t0b * rows_v[
                                e, pl.ds(288 + d * 32 + hf * 16, LANES)]
                            pay_v[e, pl.ds(72 + d * 32 + hf * 16, LANES)] = m1
                return carry3

            lax.fori_loop(0, CH // 4, edge4, 0)
            pltpu.async_copy(pay_v,
                             acc.at[dst_v.at[pl.ds(cidx * CH, CH)]],
                             ssem, add=True)

        fire_dyn(0, rows_a, gsem_a)
        lax.fori_loop(0, (SUP // CH) // 2, pair_body, 0)
        wait(rows_a, gsem_a)
        compute_dyn(SUP // CH - 1, rows_a, pay_a, ssem_a)
        return carry

    lax.fori_loop(0, NSUP, super_body, 0)
    pltpu.make_async_copy(out_ref.at[0, pl.ds(0, CH)], pay_a, ssem_a).wait()
    pltpu.make_async_copy(out_ref.at[0, pl.ds(0, CH)], pay_b, ssem_b).wait()
    plsc.subcore_barrier()

    def dump(r, carry):
        pltpu.sync_copy(acc.at[pl.ds(s * RPT + r * 64, 64)],
                        out_ref.at[c, pl.ds(s * RPT + r * 64, 64)])
        return carry
    lax.fori_loop(0, RPT // 64, dump, 0)


def _sc_scatter(table, edge_index, t):
    mesh = plsc.VectorSubcoreMesh(core_axis_name="c", subcore_axis_name="s")
    kfn = pl.kernel(
        _sc_body,
        out_type=jax.ShapeDtypeStruct((NC, NPAD, ACCW), jnp.float32),
        mesh=mesh,
        scratch_types=[
            pltpu.VMEM((SUP,), jnp.int32),
            pltpu.VMEM((SUP,), jnp.int32),
            pltpu.VMEM((4, SUP), jnp.float32),
            pltpu.VMEM((CH, TABW), jnp.float32),
            pltpu.VMEM((CH, TABW), jnp.float32),
            pltpu.VMEM((CH, ACCW), jnp.float32),
            pltpu.VMEM((CH, ACCW), jnp.float32),
            pltpu.VMEM_SHARED((NPAD, ACCW), jnp.float32),
            pltpu.SemaphoreType.DMA,
            pltpu.SemaphoreType.DMA,
            pltpu.SemaphoreType.DMA,
            pltpu.SemaphoreType.DMA,
        ],
        compiler_params=pltpu.CompilerParams(use_tc_tiling_on_sc=False),
    )
    return kfn(table, edge_index, t)


def _fin_body(acc_ref, h_ref, wsi_ref, g0, b0, g1, o_ref):
    sums = acc_ref[0, :N_, :] + acc_ref[1, :N_, :]
    cnt = jnp.maximum(sums[:, 0:1], 1.0)
    agg = sums[:, 8:168] / cnt
    ri = lax.broadcasted_iota(jnp.int32, (96, 96), 0)
    ci = lax.broadcasted_iota(jnp.int32, (96, 96), 1)
    P = ((ri % 32) * 3 + ri // 32 == ci).astype(jnp.float32)
    agg1 = jnp.dot(agg[:, 64:160], P, preferred_element_type=jnp.float32)
    agg = jnp.concatenate([agg[:, :64], agg1], axis=1)
    out = agg + jnp.dot(h_ref[...], wsi_ref[...],
                        preferred_element_type=jnp.float32)
    sc = out[:, :MUL0_]
    mu = jnp.mean(sc, axis=0, keepdims=True)
    xc = sc - mu
    var = jnp.mean(xc * xc, axis=0, keepdims=True)
    scn = xc * lax.rsqrt(var + 1e-5) * g0[...] + b0[...]
    v = out[:, MUL0_:]
    colsum = jnp.sum(v * v, axis=0, keepdims=True)
    r = lax.broadcasted_iota(jnp.int32, (96, 32), 0)
    cix = lax.broadcasted_iota(jnp.int32, (96, 32), 1)
    S = (r // 3 == cix).astype(jnp.float32)
    fn = jnp.dot(colsum, S, preferred_element_type=jnp.float32) / N_
    scale32 = g1[...] * lax.rsqrt(fn + 1e-5)
    r2 = lax.broadcasted_iota(jnp.int32, (32, 96), 0)
    c2 = lax.broadcasted_iota(jnp.int32, (32, 96), 1)
    S2 = (c2 // 3 == r2).astype(jnp.float32)
    scale96 = jnp.dot(scale32, S2, preferred_element_type=jnp.float32)
    vout = v * scale96
    o_ref[...] = jnp.concatenate([scn, vout], axis=1) + h_ref[...]


def _finish(acc, h, wsi, g0, b0, g1):
    return pl.pallas_call(
        _fin_body,
        in_specs=[
            pl.BlockSpec((NC, NPAD, ACCW), lambda: (0, 0, 0)),

            pl.BlockSpec((N_, 160), lambda: (0, 0)),
            pl.BlockSpec((160, 160), lambda: (0, 0)),
            pl.BlockSpec((1, 64), lambda: (0, 0)),
            pl.BlockSpec((1, 64), lambda: (0, 0)),
            pl.BlockSpec((1, 32), lambda: (0, 0)),
        ],
        out_specs=pl.BlockSpec((N_, 160), lambda: (0, 0)),
        out_shape=jax.ShapeDtypeStruct((N_, 160), jnp.float32),
    )(acc, h, wsi, g0.reshape(1, 64), b0.reshape(1, 64), g1.reshape(1, 32))


def _weights(p):
    A1 = 1.0 / np.sqrt(MUL0_)
    A2 = 1.0 / np.sqrt(MUL0_)
    A3 = 1.0 / np.sqrt(MUL1_)
    A4 = 1.0 / np.sqrt(MUL1_ * 3.0)
    eye3 = jnp.eye(3, dtype=jnp.float32)
    zb = A4 * jnp.einsum('de,uj->udej', eye3, p['W4']).reshape(96, 192)
    y2b = A2 * p['W2']
    y3b = A3 * jnp.einsum('uk,de->udek', p['W3'], eye3).reshape(96, 96)
    top = jnp.concatenate(
        [A1 * p['W1'], jnp.zeros((64, 192), jnp.float32), y2b,
         jnp.zeros((64, 96), jnp.float32)], axis=1)
    bot = jnp.concatenate(
        [jnp.zeros((96, 64), jnp.float32), zb,
         jnp.zeros((96, 32), jnp.float32), y3b], axis=1)
    wbig = jnp.concatenate([top, bot], axis=0)
    ws1 = jnp.einsum('uk,de->udke', p['Ws1'], eye3).reshape(96, 96)
    wsi = jnp.block(
        [[p['Ws0'] / np.sqrt(MUL0_), jnp.zeros((64, 96), jnp.float32)],
         [jnp.zeros((96, 64), jnp.float32), ws1 / np.sqrt(MUL1_)]])
    return wbig, wsi


@jax.jit
def kernel(h, edge_index, edge_sh, edge_features, params):
    wbig, wsi = _weights(params)
    table = _build_table(h, wbig)
    t = _edge_t(edge_features, edge_sh, params)
    acc = _sc_scatter(table, edge_index, t)
    return _finish(acc, h, wsi, params['g0'], params['b0'], params['g1'])


# speedup vs baseline: 2.8751x; 1.0081x over previous
import functools

import jax
import jax.numpy as jnp
import numpy as np
from jax import lax
from jax.experimental import pallas as pl
from jax.experimental.pallas import tpu as pltpu
from jax.experimental.pallas import tpu_sc as plsc

MUL0_ = 64
MUL1_ = 32
N_ = 10000
E_ = 320000
TABW = 384
ACCW = 168
NC = 2
NS = 16
LANES = 16
EPT = E_ // (NC * NS)
CH = 16
SUP = 400
NSUP = EPT // SUP
NPAD = 10240
RPT = NPAD // NS


def _table_body(h_ref, w_ref, o_ref):
    o_ref[...] = jnp.dot(h_ref[...], w_ref[...],
                         preferred_element_type=jnp.float32)


def _build_table(h, wbig):
    return pl.pallas_call(
        _table_body,
        grid=(2,),
        in_specs=[
            pl.BlockSpec((N_ // 2, 160), lambda i: (i, 0)),
            pl.BlockSpec((160, TABW), lambda i: (0, 0)),
        ],
        out_specs=pl.BlockSpec((N_ // 2, TABW), lambda i: (i, 0)),
        out_shape=jax.ShapeDtypeStruct((N_, TABW), jnp.float32),
    )(h, wbig)


def _silu(x):
    return x * (1.0 / (1.0 + jnp.exp(-x)))


def _edge_body(ef_ref, sh_ref, w1, b1, w2, b2, w3, b3, t_ref):
    f = _silu(jnp.dot(ef_ref[...], w1[...],
                      preferred_element_type=jnp.float32) + b1[...])
    f = _silu(jnp.dot(f, w2[...],
                      preferred_element_type=jnp.float32) + b2[...])
    s = jnp.dot(f, w3[...], preferred_element_type=jnp.float32) + b3[...]
    t_ref[...] = (s * sh_ref[...]).T


def _edge_t(ef, esh, p):
    B = 6400
    return pl.pallas_call(
        _edge_body,
        grid=(E_ // B,),
        in_specs=[
            pl.BlockSpec((B, 16), lambda i: (i, 0)),
            pl.BlockSpec((B, 4), lambda i: (i, 0)),
            pl.BlockSpec((16, 64), lambda i: (0, 0)),
            pl.BlockSpec((1, 64), lambda i: (0, 0)),
            pl.BlockSpec((64, 64), lambda i: (0, 0)),
            pl.BlockSpec((1, 64), lambda i: (0, 0)),
            pl.BlockSpec((64, 1), lambda i: (0, 0)),
            pl.BlockSpec((1, 1), lambda i: (0, 0)),
        ],
        out_specs=pl.BlockSpec((4, B), lambda i: (0, i)),
        out_shape=jax.ShapeDtypeStruct((4, E_), jnp.float32),
    )(ef, esh, p['mw1'], p['mb1'].reshape(1, 64), p['mw2'],
      p['mb2'].reshape(1, 64), p['mw3'], p['mb3'].reshape(1, 1))


def _lane_splat(vec, e):
    idx = jnp.full((LANES, 1), e, jnp.int32)
    return lax.gather(
        vec, idx,
        lax.GatherDimensionNumbers(offset_dims=(), collapsed_slice_dims=(0,),
                                   start_index_map=(0,)),
        (1,), mode=lax.GatherScatterMode.PROMISE_IN_BOUNDS)


def _sc_body(tab_ref, ei_ref, t_ref, out_ref,
             src_v, dst_v, t_v, rows_a, rows_b, pay_a, pay_b, acc,
             gsem_a, gsem_b, ssem_a, ssem_b):
    c = lax.axis_index("c")
    s = lax.axis_index("s")
    wid = s * NC + c
    zvec = jnp.zeros((LANES,), jnp.float32)
    iota16 = lax.broadcasted_iota(jnp.int32, (LANES,), 0)

    for pv in (pay_a, pay_b):
        for e in range(CH):
            for cc in range(ACCW // LANES):
                pv[e, pl.ds(cc * LANES, LANES)] = zvec
            pv[e, pl.ds(ACCW - LANES, LANES)] = zvec

    def zcp(j, carry):
        pltpu.async_copy(pay_a, acc.at[pl.ds(s * RPT + j * CH, CH)], gsem_a)
        return carry
    lax.fori_loop(0, RPT // CH, zcp, 0)

    def zdr(j, carry):
        pltpu.make_async_copy(out_ref.at[0, pl.ds(0, CH)], pay_a,
                              gsem_a).wait()
        return carry
    lax.fori_loop(0, RPT // CH, zdr, 0)
    plsc.subcore_barrier()

    dst_v[pl.ds(0, LANES)] = iota16 + s * RPT
    pltpu.async_copy(pay_a, acc.at[dst_v.at[pl.ds(0, CH)]], ssem_a, add=True)
    pltpu.async_copy(pay_b, acc.at[dst_v.at[pl.ds(0, CH)]], ssem_b, add=True)

    cntv = jnp.where(iota16 == 0, 1.0, 0.0).astype(jnp.float32)
    base_e = wid * EPT

    def wait(rows_v, sem):
        pltpu.make_async_copy(tab_ref.at[pl.ds(0, CH)], rows_v, sem).wait()

    def super_body(sp, carry):
        eb = base_e + sp * SUP
        pltpu.sync_copy(ei_ref.at[0, pl.ds(eb, SUP)], src_v)
        pltpu.sync_copy(ei_ref.at[1, pl.ds(eb, SUP)], dst_v)
        pltpu.sync_copy(t_ref.at[:, pl.ds(eb, SUP)], t_v)

        def fire_dyn(cidx, rows_v, sem):
            pltpu.async_copy(
                tab_ref.at[src_v.at[pl.ds(cidx * CH, CH)]], rows_v, sem)

        def pair_body(j, carry2):
            c0 = 2 * j
            fire_dyn(c0 + 1, rows_b, gsem_b)
            wait(rows_a, gsem_a)
            compute_dyn(c0, rows_a, pay_a, ssem_a)
            fire_dyn(c0 + 2, rows_a, gsem_a)
            wait(rows_b, gsem_b)
            compute_dyn(c0 + 1, rows_b, pay_b, ssem_b)
            return carry2

        def compute_dyn(cidx, rows_v, pay_v, ssem):
            tb = cidx * CH
            tv0 = t_v[0, pl.ds(tb, LANES)]
            tv1 = t_v[1, pl.ds(tb, LANES)]
            tv2 = t_v[2, pl.ds(tb, LANES)]
            tv3 = t_v[3, pl.ds(tb, LANES)]
            pltpu.make_async_copy(out_ref.at[0, pl.ds(0, CH)], pay_v,
                                  ssem).wait()

            def edge4(it, carry3):
                for k in range(8):
                    e = it * 8 + k
                    t0b = _lane_splat(tv0, e)
                    t1b = _lane_splat(tv1, e)
                    t2b = _lane_splat(tv2, e)
                    t3b = _lane_splat(tv3, e)
                    tdb = (t1b, t2b, t3b)
                    pay_v[e, pl.ds(0, LANES)] = cntv
                    for j in range(4):
                        a = (t0b * rows_v[e, pl.ds(j * 16, LANES)]
                             + t1b * rows_v[e, pl.ds(64 + j * 16, LANES)]) \
                            + (t2b * rows_v[e, pl.ds(128 + j * 16, LANES)]
                               + t3b * rows_v[e, pl.ds(192 + j * 16, LANES)])
                        pay_v[e, pl.ds(8 + j * 16, LANES)] = a
                    y2h = (rows_v[e, pl.ds(256, LANES)],
                           rows_v[e, pl.ds(272, LANES)])
                    for d in range(3):
                        for hf in range(2):
                            m1 = tdb[d] * y2h[hf] + t0b * rows_v[
                                e, pl.ds(288 + d * 32 + hf * 16, LANES)]
                            pay_v[e, pl.ds(72 + d * 32 + hf * 16, LANES)] = m1
                return carry3

            lax.fori_loop(0, CH // 8, edge4, 0)
            pltpu.async_copy(pay_v,
                             acc.at[dst_v.at[pl.ds(cidx * CH, CH)]],
                             ssem, add=True)

        fire_dyn(0, rows_a, gsem_a)
        lax.fori_loop(0, (SUP // CH) // 2, pair_body, 0)
        wait(rows_a, gsem_a)
        compute_dyn(SUP // CH - 1, rows_a, pay_a, ssem_a)
        return carry

    lax.fori_loop(0, NSUP, super_body, 0)
    pltpu.make_async_copy(out_ref.at[0, pl.ds(0, CH)], pay_a, ssem_a).wait()
    pltpu.make_async_copy(out_ref.at[0, pl.ds(0, CH)], pay_b, ssem_b).wait()
    plsc.subcore_barrier()

    def dump(r, carry):
        pltpu.sync_copy(acc.at[pl.ds(s * RPT + r * 64, 64)],
                        out_ref.at[c, pl.ds(s * RPT + r * 64, 64)])
        return carry
    lax.fori_loop(0, RPT // 64, dump, 0)


def _sc_scatter(table, edge_index, t):
    mesh = plsc.VectorSubcoreMesh(core_axis_name="c", subcore_axis_name="s")
    kfn = pl.kernel(
        _sc_body,
        out_type=jax.ShapeDtypeStruct((NC, NPAD, ACCW), jnp.float32),
        mesh=mesh,
        scratch_types=[
            pltpu.VMEM((SUP,), jnp.int32),
            pltpu.VMEM((SUP,), jnp.int32),
            pltpu.VMEM((4, SUP), jnp.float32),
            pltpu.VMEM((CH, TABW), jnp.float32),
            pltpu.VMEM((CH, TABW), jnp.float32),
            pltpu.VMEM((CH, ACCW), jnp.float32),
            pltpu.VMEM((CH, ACCW), jnp.float32),
            pltpu.VMEM_SHARED((NPAD, ACCW), jnp.float32),
            pltpu.SemaphoreType.DMA,
            pltpu.SemaphoreType.DMA,
            pltpu.SemaphoreType.DMA,
            pltpu.SemaphoreType.DMA,
        ],
        compiler_params=pltpu.CompilerParams(use_tc_tiling_on_sc=False),
    )
    return kfn(table, edge_index, t)


def _fin_body(acc_ref, h_ref, wsi_ref, g0, b0, g1, o_ref):
    sums = acc_ref[0, :N_, :] + acc_ref[1, :N_, :]
    cnt = jnp.maximum(sums[:, 0:1], 1.0)
    agg = sums[:, 8:168] / cnt
    ri = lax.broadcasted_iota(jnp.int32, (96, 96), 0)
    ci = lax.broadcasted_iota(jnp.int32, (96, 96), 1)
    P = ((ri % 32) * 3 + ri // 32 == ci).astype(jnp.float32)
    agg1 = jnp.dot(agg[:, 64:160], P, preferred_element_type=jnp.float32)
    agg = jnp.concatenate([agg[:, :64], agg1], axis=1)
    out = agg + jnp.dot(h_ref[...], wsi_ref[...],
                        preferred_element_type=jnp.float32)
    sc = out[:, :MUL0_]
    mu = jnp.mean(sc, axis=0, keepdims=True)
    xc = sc - mu
    var = jnp.mean(xc * xc, axis=0, keepdims=True)
    scn = xc * lax.rsqrt(var + 1e-5) * g0[...] + b0[...]
    v = out[:, MUL0_:]
    colsum = jnp.sum(v * v, axis=0, keepdims=True)
    r = lax.broadcasted_iota(jnp.int32, (96, 32), 0)
    cix = lax.broadcasted_iota(jnp.int32, (96, 32), 1)
    S = (r // 3 == cix).astype(jnp.float32)
    fn = jnp.dot(colsum, S, preferred_element_type=jnp.float32) / N_
    scale32 = g1[...] * lax.rsqrt(fn + 1e-5)
    r2 = lax.broadcasted_iota(jnp.int32, (32, 96), 0)
    c2 = lax.broadcasted_iota(jnp.int32, (32, 96), 1)
    S2 = (c2 // 3 == r2).astype(jnp.float32)
    scale96 = jnp.dot(scale32, S2, preferred_element_type=jnp.float32)
    vout = v * scale96
    o_ref[...] = jnp.concatenate([scn, vout], axis=1) + h_ref[...]


def _finish(acc, h, wsi, g0, b0, g1):
    return pl.pallas_call(
        _fin_body,
        in_specs=[
            pl.BlockSpec((NC, NPAD, ACCW), lambda: (0, 0, 0)),

            pl.BlockSpec((N_, 160), lambda: (0, 0)),
            pl.BlockSpec((160, 160), lambda: (0, 0)),
            pl.BlockSpec((1, 64), lambda: (0, 0)),
            pl.BlockSpec((1, 64), lambda: (0, 0)),
            pl.BlockSpec((1, 32), lambda: (0, 0)),
        ],
        out_specs=pl.BlockSpec((N_, 160), lambda: (0, 0)),
        out_shape=jax.ShapeDtypeStruct((N_, 160), jnp.float32),
    )(acc, h, wsi, g0.reshape(1, 64), b0.reshape(1, 64), g1.reshape(1, 32))


def _weights(p):
    A1 = 1.0 / np.sqrt(MUL0_)
    A2 = 1.0 / np.sqrt(MUL0_)
    A3 = 1.0 / np.sqrt(MUL1_)
    A4 = 1.0 / np.sqrt(MUL1_ * 3.0)
    eye3 = jnp.eye(3, dtype=jnp.float32)
    zb = A4 * jnp.einsum('de,uj->udej', eye3, p['W4']).reshape(96, 192)
    y2b = A2 * p['W2']
    y3b = A3 * jnp.einsum('uk,de->udek', p['W3'], eye3).reshape(96, 96)
    top = jnp.concatenate(
        [A1 * p['W1'], jnp.zeros((64, 192), jnp.float32), y2b,
         jnp.zeros((64, 96), jnp.float32)], axis=1)
    bot = jnp.concatenate(
        [jnp.zeros((96, 64), jnp.float32), zb,
         jnp.zeros((96, 32), jnp.float32), y3b], axis=1)
    wbig = jnp.concatenate([top, bot], axis=0)
    ws1 = jnp.einsum('uk,de->udke', p['Ws1'], eye3).reshape(96, 96)
    wsi = jnp.block(
        [[p['Ws0'] / np.sqrt(MUL0_), jnp.zeros((64, 96), jnp.float32)],
         [jnp.zeros((96, 64), jnp.float32), ws1 / np.sqrt(MUL1_)]])
    return wbig, wsi


@jax.jit
def kernel(h, edge_index, edge_sh, edge_features, params):
    wbig, wsi = _weights(params)
    table = _build_table(h, wbig)
    t = _edge_t(edge_features, edge_sh, params)
    acc = _sc_scatter(table, edge_index, t)
    return _finish(acc, h, wsi, params['g0'], params['b0'], params['g1'])


# speedup vs baseline: 2.9313x; 1.0195x over previous
import functools

import jax
import jax.numpy as jnp
import numpy as np
from jax import lax
from jax.experimental import pallas as pl
from jax.experimental.pallas import tpu as pltpu
from jax.experimental.pallas import tpu_sc as plsc

MUL0_ = 64
MUL1_ = 32
N_ = 10000
E_ = 320000
TABW = 384
ACCW = 168
NC = 2
NS = 16
LANES = 16
EPT = E_ // (NC * NS)
CH = 16
SUP = 400
NSUP = EPT // SUP
NPAD = 10240
RPT = NPAD // NS


def _table_body(h_ref, w_ref, o_ref):
    o_ref[...] = jnp.dot(h_ref[...], w_ref[...],
                         preferred_element_type=jnp.float32)


def _build_table(h, wbig):
    return pl.pallas_call(
        _table_body,
        grid=(2,),
        in_specs=[
            pl.BlockSpec((N_ // 2, 160), lambda i: (i, 0)),
            pl.BlockSpec((160, TABW), lambda i: (0, 0)),
        ],
        out_specs=pl.BlockSpec((N_ // 2, TABW), lambda i: (i, 0)),
        out_shape=jax.ShapeDtypeStruct((N_, TABW), jnp.float32),
    )(h, wbig)


def _silu(x):
    return x * (1.0 / (1.0 + jnp.exp(-x)))


def _edge_body(ef_ref, sh_ref, w1, b1, w2, b2, w3, b3, t_ref):
    f = _silu(jnp.dot(ef_ref[...], w1[...],
                      preferred_element_type=jnp.float32) + b1[...])
    f = _silu(jnp.dot(f, w2[...],
                      preferred_element_type=jnp.float32) + b2[...])
    s = jnp.dot(f, w3[...], preferred_element_type=jnp.float32) + b3[...]
    t_ref[...] = (s * sh_ref[...]).T


def _edge_t(ef, esh, p):
    B = 6400
    return pl.pallas_call(
        _edge_body,
        grid=(E_ // B,),
        in_specs=[
            pl.BlockSpec((B, 16), lambda i: (i, 0)),
            pl.BlockSpec((B, 4), lambda i: (i, 0)),
            pl.BlockSpec((16, 64), lambda i: (0, 0)),
            pl.BlockSpec((1, 64), lambda i: (0, 0)),
            pl.BlockSpec((64, 64), lambda i: (0, 0)),
            pl.BlockSpec((1, 64), lambda i: (0, 0)),
            pl.BlockSpec((64, 1), lambda i: (0, 0)),
            pl.BlockSpec((1, 1), lambda i: (0, 0)),
        ],
        out_specs=pl.BlockSpec((4, B), lambda i: (0, i)),
        out_shape=jax.ShapeDtypeStruct((4, E_), jnp.float32),
    )(ef, esh, p['mw1'], p['mb1'].reshape(1, 64), p['mw2'],
      p['mb2'].reshape(1, 64), p['mw3'], p['mb3'].reshape(1, 1))


def _lane_splat(vec, e):
    idx = jnp.full((LANES, 1), e, jnp.int32)
    return lax.gather(
        vec, idx,
        lax.GatherDimensionNumbers(offset_dims=(), collapsed_slice_dims=(0,),
                                   start_index_map=(0,)),
        (1,), mode=lax.GatherScatterMode.PROMISE_IN_BOUNDS)


def _sc_body(tab_ref, ei_ref, t_ref, out_ref,
             src_v, dst_v, t_v, rows_a, rows_b, pay_a, pay_b, acc,
             gsem_a, gsem_b, ssem_a, ssem_b):
    c = lax.axis_index("c")
    s = lax.axis_index("s")
    wid = s * NC + c
    zvec = jnp.zeros((LANES,), jnp.float32)
    iota16 = lax.broadcasted_iota(jnp.int32, (LANES,), 0)

    for pv in (pay_a, pay_b):
        for e in range(CH):
            for cc in range(ACCW // LANES):
                pv[e, pl.ds(cc * LANES, LANES)] = zvec
            pv[e, pl.ds(ACCW - LANES, LANES)] = zvec

    def zcp(j, carry):
        pltpu.async_copy(pay_a, acc.at[pl.ds(s * RPT + j * CH, CH)], gsem_a)
        return carry
    lax.fori_loop(0, RPT // CH, zcp, 0)

    def zdr(j, carry):
        pltpu.make_async_copy(out_ref.at[0, pl.ds(0, CH)], pay_a,
                              gsem_a).wait()
        return carry
    lax.fori_loop(0, RPT // CH, zdr, 0)
    plsc.subcore_barrier()

    dst_v[pl.ds(0, LANES)] = iota16 + s * RPT
    pltpu.async_copy(pay_a, acc.at[dst_v.at[pl.ds(0, CH)]], ssem_a, add=True)
    pltpu.async_copy(pay_b, acc.at[dst_v.at[pl.ds(0, CH)]], ssem_b, add=True)

    cntv = jnp.where(iota16 == 0, 1.0, 0.0).astype(jnp.float32)
    base_e = wid * EPT

    def wait(rows_v, sem):
        pltpu.make_async_copy(tab_ref.at[pl.ds(0, CH)], rows_v, sem).wait()

    def super_body(sp, carry):
        eb = base_e + sp * SUP
        pltpu.async_copy(ei_ref.at[0, pl.ds(eb, SUP)], src_v, gsem_b)
        pltpu.async_copy(ei_ref.at[1, pl.ds(eb, SUP)], dst_v, gsem_b)
        pltpu.async_copy(t_ref.at[:, pl.ds(eb, SUP)], t_v, gsem_b)
        pltpu.make_async_copy(ei_ref.at[0, pl.ds(0, SUP)], src_v,
                              gsem_b).wait()
        pltpu.make_async_copy(ei_ref.at[0, pl.ds(0, SUP)], dst_v,
                              gsem_b).wait()
        pltpu.make_async_copy(t_ref.at[:, pl.ds(0, SUP)], t_v,
                              gsem_b).wait()

        def fire_dyn(cidx, rows_v, sem):
            pltpu.async_copy(
                tab_ref.at[src_v.at[pl.ds(cidx * CH, CH)]], rows_v, sem)

        def pair_body(j, carry2):
            c0 = 2 * j
            fire_dyn(c0 + 1, rows_b, gsem_b)
            wait(rows_a, gsem_a)
            compute_dyn(c0, rows_a, pay_a, ssem_a)
            fire_dyn(c0 + 2, rows_a, gsem_a)
            wait(rows_b, gsem_b)
            compute_dyn(c0 + 1, rows_b, pay_b, ssem_b)
            return carry2

        def compute_dyn(cidx, rows_v, pay_v, ssem):
            tb = cidx * CH
            tv0 = t_v[0, pl.ds(tb, LANES)]
            tv1 = t_v[1, pl.ds(tb, LANES)]
            tv2 = t_v[2, pl.ds(tb, LANES)]
            tv3 = t_v[3, pl.ds(tb, LANES)]
            pltpu.make_async_copy(out_ref.at[0, pl.ds(0, CH)], pay_v,
                                  ssem).wait()

            def edge4(it, carry3):
                for k in range(8):
                    e = it * 8 + k
                    t0b = _lane_splat(tv0, e)
                    t1b = _lane_splat(tv1, e)
                    t2b = _lane_splat(tv2, e)
                    t3b = _lane_splat(tv3, e)
                    tdb = (t1b, t2b, t3b)
                    pay_v[e, pl.ds(0, LANES)] = cntv
                    for j in range(4):
                        a = (t0b * rows_v[e, pl.ds(j * 16, LANES)]
                             + t1b * rows_v[e, pl.ds(64 + j * 16, LANES)]) \
                            + (t2b * rows_v[e, pl.ds(128 + j * 16, LANES)]
                               + t3b * rows_v[e, pl.ds(192 + j * 16, LANES)])
                        pay_v[e, pl.ds(8 + j * 16, LANES)] = a
                    y2h = (rows_v[e, pl.ds(256, LANES)],
                           rows_v[e, pl.ds(272, LANES)])
                    for d in range(3):
                        for hf in range(2):
                            m1 = tdb[d] * y2h[hf] + t0b * rows_v[
                                e, pl.ds(288 + d * 32 + hf * 16, LANES)]
                            pay_v[e, pl.ds(72 + d * 32 + hf * 16, LANES)] = m1
                return carry3

            lax.fori_loop(0, CH // 8, edge4, 0)
            pltpu.async_copy(pay_v,
                             acc.at[dst_v.at[pl.ds(cidx * CH, CH)]],
                             ssem, add=True)

        fire_dyn(0, rows_a, gsem_a)
        lax.fori_loop(0, (SUP // CH) // 2, pair_body, 0)
        wait(rows_a, gsem_a)
        compute_dyn(SUP // CH - 1, rows_a, pay_a, ssem_a)
        return carry

    lax.fori_loop(0, NSUP, super_body, 0)
    pltpu.make_async_copy(out_ref.at[0, pl.ds(0, CH)], pay_a, ssem_a).wait()
    pltpu.make_async_copy(out_ref.at[0, pl.ds(0, CH)], pay_b, ssem_b).wait()
    plsc.subcore_barrier()

    def dump(r, carry):
        pltpu.sync_copy(acc.at[pl.ds(s * RPT + r * 64, 64)],
                        out_ref.at[c, pl.ds(s * RPT + r * 64, 64)])
        return carry
    lax.fori_loop(0, RPT // 64, dump, 0)


def _sc_scatter(table, edge_index, t):
    mesh = plsc.VectorSubcoreMesh(core_axis_name="c", subcore_axis_name="s")
    kfn = pl.kernel(
        _sc_body,
        out_type=jax.ShapeDtypeStruct((NC, NPAD, ACCW), jnp.float32),
        mesh=mesh,
        scratch_types=[
            pltpu.VMEM((SUP,), jnp.int32),
            pltpu.VMEM((SUP,), jnp.int32),
            pltpu.VMEM((4, SUP), jnp.float32),
            pltpu.VMEM((CH, TABW), jnp.float32),
            pltpu.VMEM((CH, TABW), jnp.float32),
            pltpu.VMEM((CH, ACCW), jnp.float32),
            pltpu.VMEM((CH, ACCW), jnp.float32),
            pltpu.VMEM_SHARED((NPAD, ACCW), jnp.float32),
            pltpu.SemaphoreType.DMA,
            pltpu.SemaphoreType.DMA,
            pltpu.SemaphoreType.DMA,
            pltpu.SemaphoreType.DMA,
        ],
        compiler_params=pltpu.CompilerParams(use_tc_tiling_on_sc=False),
    )
    return kfn(table, edge_index, t)


def _fin_body(acc_ref, h_ref, wsi_ref, g0, b0, g1, o_ref):
    sums = acc_ref[0, :N_, :] + acc_ref[1, :N_, :]
    cnt = jnp.maximum(sums[:, 0:1], 1.0)
    agg = sums[:, 8:168] / cnt
    ri = lax.broadcasted_iota(jnp.int32, (96, 96), 0)
    ci = lax.broadcasted_iota(jnp.int32, (96, 96), 1)
    P = ((ri % 32) * 3 + ri // 32 == ci).astype(jnp.float32)
    agg1 = jnp.dot(agg[:, 64:160], P, preferred_element_type=jnp.float32)
    agg = jnp.concatenate([agg[:, :64], agg1], axis=1)
    out = agg + jnp.dot(h_ref[...], wsi_ref[...],
                        preferred_element_type=jnp.float32)
    sc = out[:, :MUL0_]
    mu = jnp.mean(sc, axis=0, keepdims=True)
    xc = sc - mu
    var = jnp.mean(xc * xc, axis=0, keepdims=True)
    scn = xc * lax.rsqrt(var + 1e-5) * g0[...] + b0[...]
    v = out[:, MUL0_:]
    colsum = jnp.sum(v * v, axis=0, keepdims=True)
    r = lax.broadcasted_iota(jnp.int32, (96, 32), 0)
    cix = lax.broadcasted_iota(jnp.int32, (96, 32), 1)
    S = (r // 3 == cix).astype(jnp.float32)
    fn = jnp.dot(colsum, S, preferred_element_type=jnp.float32) / N_
    scale32 = g1[...] * lax.rsqrt(fn + 1e-5)
    r2 = lax.broadcasted_iota(jnp.int32, (32, 96), 0)
    c2 = lax.broadcasted_iota(jnp.int32, (32, 96), 1)
    S2 = (c2 // 3 == r2).astype(jnp.float32)
    scale96 = jnp.dot(scale32, S2, preferred_element_type=jnp.float32)
    vout = v * scale96
    o_ref[...] = jnp.concatenate([scn, vout], axis=1) + h_ref[...]


def _finish(acc, h, wsi, g0, b0, g1):
    return pl.pallas_call(
        _fin_body,
        in_specs=[
            pl.BlockSpec((NC, NPAD, ACCW), lambda: (0, 0, 0)),

            pl.BlockSpec((N_, 160), lambda: (0, 0)),
            pl.BlockSpec((160, 160), lambda: (0, 0)),
            pl.BlockSpec((1, 64), lambda: (0, 0)),
            pl.BlockSpec((1, 64), lambda: (0, 0)),
            pl.BlockSpec((1, 32), lambda: (0, 0)),
        ],
        out_specs=pl.BlockSpec((N_, 160), lambda: (0, 0)),
        out_shape=jax.ShapeDtypeStruct((N_, 160), jnp.float32),
    )(acc, h, wsi, g0.reshape(1, 64), b0.reshape(1, 64), g1.reshape(1, 32))


def _weights(p):
    A1 = 1.0 / np.sqrt(MUL0_)
    A2 = 1.0 / np.sqrt(MUL0_)
    A3 = 1.0 / np.sqrt(MUL1_)
    A4 = 1.0 / np.sqrt(MUL1_ * 3.0)
    eye3 = jnp.eye(3, dtype=jnp.float32)
    zb = A4 * jnp.einsum('de,uj->udej', eye3, p['W4']).reshape(96, 192)
    y2b = A2 * p['W2']
    y3b = A3 * jnp.einsum('uk,de->udek', p['W3'], eye3).reshape(96, 96)
    top = jnp.concatenate(
        [A1 * p['W1'], jnp.zeros((64, 192), jnp.float32), y2b,
         jnp.zeros((64, 96), jnp.float32)], axis=1)
    bot = jnp.concatenate(
        [jnp.zeros((96, 64), jnp.float32), zb,
         jnp.zeros((96, 32), jnp.float32), y3b], axis=1)
    wbig = jnp.concatenate([top, bot], axis=0)
    ws1 = jnp.einsum('uk,de->udke', p['Ws1'], eye3).reshape(96, 96)
    wsi = jnp.block(
        [[p['Ws0'] / np.sqrt(MUL0_), jnp.zeros((64, 96), jnp.float32)],
         [jnp.zeros((96, 64), jnp.float32), ws1 / np.sqrt(MUL1_)]])
    return wbig, wsi


@jax.jit
def kernel(h, edge_index, edge_sh, edge_features, params):
    wbig, wsi = _weights(params)
    table = _build_table(h, wbig)
    t = _edge_t(edge_features, edge_sh, params)
    acc = _sc_scatter(table, edge_index, t)
    return _finish(acc, h, wsi, params['g0'], params['b0'], params['g1'])
